# baseline (speedup 1.0000x reference)
import functools

import jax
import jax.numpy as jnp
from jax import lax
from jax.experimental import pallas as pl
from jax.experimental.pallas import tpu as pltpu
from jax.experimental.pallas import tpu_sc as plsc

N = 10000
E = 320000
D = 128
NC = 2
NS = 16
NW = NC * NS
EPW = E // NW
C = 80
NCH = EPW // C
PD = 4
NCHP = NCH + 3
RB = 10
TCB = N // RB
ZR = 624
REM_OFF = NS * ZR
REM = N - REM_OFF


def _zst_body(h_ref, wfc_ref, wa_ref, z_ref, st_ref):
    h = h_ref[...]
    z = lax.dot_general(h, wfc_ref[...], (((1,), (1,)), ((), ())),
                        preferred_element_type=jnp.float32)
    z_ref[...] = z
    st_ref[...] = lax.dot_general(z, wa_ref[...], (((1,), (1,)), ((), ())),
                                  preferred_element_type=jnp.float32)


def _zst(h, wfc, wa2):
    return pl.pallas_call(
        _zst_body,
        grid=(RB,),
        in_specs=[
            pl.BlockSpec((TCB, D), lambda i: (i, 0)),
            pl.BlockSpec((D, D), lambda i: (0, 0)),
            pl.BlockSpec((2, D), lambda i: (0, 0)),
        ],
        out_specs=[
            pl.BlockSpec((TCB, D), lambda i: (i, 0)),
            pl.BlockSpec((TCB, 2), lambda i: (i, 0)),
        ],
        out_shape=[
            jax.ShapeDtypeStruct((N, D), jnp.float32),
            jax.ShapeDtypeStruct((N, 2), jnp.float32),
        ],
    )(h, wfc, wa2)


_SC_MESH = plsc.VectorSubcoreMesh(core_axis_name="c", subcore_axis_name="s")


@functools.partial(
    pl.kernel,
    mesh=_SC_MESH,
    out_type=(
        jax.ShapeDtypeStruct((NC, N, D), jnp.float32),
        jax.ShapeDtypeStruct((N,), jnp.float32),
        jax.ShapeDtypeStruct((N,), jnp.float32),
    ),
    scratch_types=(
        [pltpu.VMEM((1, C), jnp.int32)] * PD
        + [pltpu.VMEM((1, C), jnp.int32)] * PD
        + [pltpu.VMEM((C,), jnp.float32)] * PD
        + [pltpu.VMEM((C,), jnp.float32)] * PD
        + [pltpu.VMEM((C,), jnp.float32)] * PD
        + [pltpu.VMEM((C, D), jnp.float32)] * PD
        + [
            pltpu.VMEM_SHARED((N, D), jnp.float32),
            pltpu.VMEM_SHARED((N,), jnp.float32),
        ]
        + [pltpu.SemaphoreType.DMA] * (5 * PD)
    ),
    compiler_params=pltpu.CompilerParams(needs_layout_passes=False),
)
def _gat_sc(src_hbm, dst_hbm, s_hbm, t_hbm, z_hbm, zrow_hbm, zden_hbm,
            pout_hbm, pden0_hbm, pden1_hbm,
            *scr):
    srcb = scr[0:PD]
    dstb = scr[PD:2 * PD]
    sb = scr[2 * PD:3 * PD]
    tb = scr[3 * PD:4 * PD]
    wb = scr[4 * PD:5 * PD]
    rows = scr[5 * PD:6 * PD]
    out_sh, den_sh = scr[6 * PD], scr[6 * PD + 1]
    semidx = scr[6 * PD + 2:6 * PD + 2 + PD]
    semz = scr[6 * PD + 2 + PD:6 * PD + 2 + 2 * PD]
    semst = scr[6 * PD + 2 + 2 * PD:6 * PD + 2 + 3 * PD]
    semr = scr[6 * PD + 2 + 3 * PD:6 * PD + 2 + 4 * PD]
    semd = scr[6 * PD + 2 + 4 * PD:6 * PD + 2 + 5 * PD]

    cid = lax.axis_index("c")
    sid = lax.axis_index("s")
    wid = cid * NS + sid

    pltpu.sync_copy(zrow_hbm.at[pl.ds(sid * ZR, ZR)],
                    out_sh.at[pl.ds(sid * ZR, ZR)])

    @pl.when(sid == NS - 1)
    def _():
        pltpu.sync_copy(zrow_hbm.at[pl.ds(REM_OFF, REM)],
                        out_sh.at[pl.ds(REM_OFF, REM)])

    @pl.when(sid == 0)
    def _():
        pltpu.sync_copy(zden_hbm, den_sh)

    plsc.subcore_barrier()

    zeros16 = jnp.zeros((16,), jnp.int32)

    def idx_copy(c, k):
        pltpu.async_copy(src_hbm.at[wid, c], srcb[k], semidx[k])
        pltpu.async_copy(dst_hbm.at[wid, c], dstb[k], semidx[k])

    def wait_idx(k):
        pltpu.make_async_copy(src_hbm.at[0, 0], srcb[k], semidx[k]).wait()
        pltpu.make_async_copy(src_hbm.at[0, 0], dstb[k], semidx[k]).wait()

    def issue_gathers(k):
        sidx = srcb[k].at[0]
        pltpu.async_copy(z_hbm.at[sidx], rows[k], semz[k])
        pltpu.async_copy(s_hbm.at[sidx], sb[k], semst[k])
        pltpu.async_copy(t_hbm.at[dstb[k].at[0]], tb[k], semst[k])

    def wait_st(k):
        pltpu.make_async_copy(s_hbm.at[pl.ds(0, C)], sb[k], semst[k]).wait()
        pltpu.make_async_copy(t_hbm.at[pl.ds(0, C)], tb[k], semst[k]).wait()

    def wait_z(k):
        pltpu.make_async_copy(z_hbm.at[pl.ds(0, C)], rows[k], semz[k]).wait()

    def compute_w(k):
        for g in range(C // 16):
            sl = pl.ds(g * 16, 16)
            e = sb[k][sl] + tb[k][sl]
            e = jnp.where(e > 0.0, e, e * 0.01)
            wb[k][sl] = jnp.exp(e)

    UNROLL = 8

    def scale_rows(k):
        def scale(j, carry):
            j0 = j * UNROLL
            for u in range(UNROLL):
                r = j0 + u
                wsplat = plsc.load_gather(wb[k], [zeros16 + r])
                for kk in range(D // 16):
                    sl = pl.ds(kk * 16, 16)
                    rows[k][r, sl] = rows[k][r, sl] * wsplat
            return carry

        lax.fori_loop(0, C // UNROLL, scale, 0)

    def issue_scatters(k):
        didx = dstb[k].at[0]
        pltpu.async_copy(rows[k], out_sh.at[didx], semr[k], add=True)
        pltpu.async_copy(wb[k], den_sh.at[didx], semd[k], add=True)

    def wait_scatters(k):
        pltpu.make_async_copy(rows[k], z_hbm.at[pl.ds(0, C)], semr[k]).wait()
        pltpu.make_async_copy(wb[k], s_hbm.at[pl.ds(0, C)], semd[k]).wait()

    def step(c, r):
        r1, r2, r3 = (r + 1) % PD, (r + 2) % PD, (r + 3) % PD
        wait_scatters(r3)
        idx_copy(c + 3, r3)
        wait_idx(r2)
        issue_gathers(r2)
        wait_st(r)
        compute_w(r)
        wait_z(r)
        scale_rows(r)
        issue_scatters(r)

    idx_copy(0, 0)
    idx_copy(1, 1)
    idx_copy(2, 2)
    wait_idx(0)
    issue_gathers(0)
    wait_idx(1)
    issue_gathers(1)
    idx_copy(3, 3)
    wait_idx(2)
    issue_gathers(2)
    wait_st(0)
    compute_w(0)
    wait_z(0)
    scale_rows(0)
    issue_scatters(0)

    def quad(q, carry):
        c0 = 4 * q + 1
        for u in range(4):
            step(c0 + u, (1 + u) % PD)
        return carry

    lax.fori_loop(0, (NCH - 1) // 4, quad, 0)

    wait_scatters(0)
    wait_st(1)
    wait_z(1)
    wait_st(2)
    wait_z(2)
    wait_idx(3)

    plsc.subcore_barrier()
    pltpu.sync_copy(out_sh.at[pl.ds(sid * ZR, ZR)],
                    pout_hbm.at[cid, pl.ds(sid * ZR, ZR)])

    @pl.when(sid == NS - 1)
    def _():
        pltpu.sync_copy(out_sh.at[pl.ds(REM_OFF, REM)],
                        pout_hbm.at[cid, pl.ds(REM_OFF, REM)])

    @pl.when(jnp.logical_and(sid == 0, cid == 0))
    def _():
        pltpu.sync_copy(den_sh, pden0_hbm)

    @pl.when(jnp.logical_and(sid == 0, cid == 1))
    def _():
        pltpu.sync_copy(den_sh, pden1_hbm)


def _fin_body(pout_ref, pd0_ref, pd1_ref, out_ref):
    p = pout_ref[0] + pout_ref[1]
    d = pd0_ref[...] + pd1_ref[...]
    out_ref[...] = p / jnp.maximum(d, 1e-16)


def _finish(pout, pd0, pd1):
    return pl.pallas_call(
        _fin_body,
        grid=(RB,),
        in_specs=[
            pl.BlockSpec((NC, TCB, D), lambda i: (0, i, 0)),
            pl.BlockSpec((TCB, 1), lambda i: (i, 0)),
            pl.BlockSpec((TCB, 1), lambda i: (i, 0)),
        ],
        out_specs=pl.BlockSpec((TCB, D), lambda i: (i, 0)),
        out_shape=jax.ShapeDtypeStruct((N, D), jnp.float32),
    )(pout, pd0, pd1)


def kernel(h, edge_index, W_fc, W_attn):
    z, st = _zst(h, W_fc, W_attn.reshape(2, D))
    pad = jnp.zeros((NW, (NCHP - NCH) * C), jnp.int32)
    src = jnp.concatenate([edge_index[0].reshape(NW, EPW), pad],
                          axis=1).reshape(NW, NCHP, 1, C)
    dst = jnp.concatenate([edge_index[1].reshape(NW, EPW), pad],
                          axis=1).reshape(NW, NCHP, 1, C)
    zrow = jnp.zeros((N, D), jnp.float32)
    zden = jnp.zeros((N,), jnp.float32)
    pout, pden0, pden1 = _gat_sc(src, dst, st[:, 0], st[:, 1], z,
                                 zrow, zden)
    return _finish(pout, pden0.reshape(N, 1), pden1.reshape(N, 1))
